# baseline (speedup 1.0000x reference)
import functools

import jax
import jax.numpy as jnp
from jax import lax
from jax.experimental import pallas as pl
from jax.experimental.pallas import tpu as pltpu
from jax.experimental.pallas import tpu_sc as plsc

ROWS, N = 128, 32768
NC, NS = 2, 16
NW = NC * NS
RPW = ROWS // NW
LANES = 16
CHUNK = N // LANES
SUB = 4
SUBLEN = CHUNK // SUB
NBINS = 256
FMASK = 0x7FFFFFFF


def _sort_body(in_hbm, out_hbm, row_f, bufa, bufb, hists):
    wid = lax.axis_index("s") * NC + lax.axis_index("c")
    lane = lax.iota(jnp.int32, LANES)
    base_idx = lane * CHUNK
    sub_base = [base_idx + j * SUBLEN for j in range(SUB)]
    ones = jnp.ones((LANES,), jnp.int32)
    zeros = jnp.zeros((LANES,), jnp.int32)

    def load_key_f32(idx):
        x = plsc.load_gather(row_f, [idx])
        u = plsc.bitcast(x, jnp.int32)
        return jnp.where(u < 0, u, u ^ FMASK)

    def store_i32(dst, pos, key):
        plsc.store_scatter(dst, [pos], key)

    def store_f32(dst, pos, key):
        v = jnp.where(key < 0, key, key ^ FMASK)
        plsc.store_scatter(dst, [pos], plsc.bitcast(v, jnp.float32))

    def one_pass(load_key, store_val, dst, shift):
        def zero_body(b, _):
            for j in range(SUB):
                hists[j][pl.ds(b * LANES, LANES)] = zeros
            return 0

        lax.fori_loop(0, NBINS, zero_body, 0)

        def hist_body(i, _):
            for j in range(SUB):
                key = load_key(sub_base[j] + i)
                d = lax.shift_right_logical(key, shift) & 0xFF
                flat = (d << 4) | lane
                plsc.addupdate_scatter(hists[j], [flat], ones)
            return 0

        lax.fori_loop(0, SUBLEN, hist_body, 0)

        def scan_body(b, carry):
            sl = pl.ds(b * LANES, LANES)
            v = [hists[j][sl] for j in range(SUB)]
            t = v[0] + v[1] + v[2] + v[3]
            o = plsc.cumsum(t) - t + carry
            for j in range(SUB):
                hists[j][sl] = o
                o = o + v[j]
            return carry + jnp.sum(t)

        lax.fori_loop(0, NBINS, scan_body, jnp.int32(0))

        def perm_body(i, _):
            for j in range(SUB):
                key = load_key(sub_base[j] + i)
                d = lax.shift_right_logical(key, shift) & 0xFF
                flat = (d << 4) | lane
                pos = plsc.load_gather(hists[j], [flat])
                plsc.addupdate_scatter(hists[j], [flat], ones)
                store_val(dst, pos, key)
            return 0

        lax.fori_loop(0, SUBLEN, perm_body, 0)

    def row_body(r, _):
        row = wid * RPW + r
        pltpu.sync_copy(in_hbm.at[row], row_f)
        one_pass(load_key_f32, store_i32, bufa, 0)
        one_pass(lambda idx: plsc.load_gather(bufa, [idx]), store_i32, bufb, 8)
        one_pass(lambda idx: plsc.load_gather(bufb, [idx]), store_i32, bufa, 16)
        one_pass(lambda idx: plsc.load_gather(bufa, [idx]), store_f32, row_f, 24)
        pltpu.sync_copy(row_f, out_hbm.at[row])
        return 0

    lax.fori_loop(0, RPW, row_body, 0)


@functools.partial(
    pl.kernel,
    out_type=jax.ShapeDtypeStruct((ROWS, N), jnp.float32),
    mesh=plsc.VectorSubcoreMesh(core_axis_name="c", subcore_axis_name="s"),
    scratch_types=[
        pltpu.VMEM((N,), jnp.float32),
        pltpu.VMEM((N,), jnp.int32),
        pltpu.VMEM((N,), jnp.int32),
        [pltpu.VMEM((NBINS * LANES,), jnp.int32) for _ in range(SUB)],
    ],
    compiler_params=pltpu.CompilerParams(needs_layout_passes=False),
)
def _sort_kernel(in_hbm, out_hbm, row_f, bufa, bufb, hists):
    _sort_body(in_hbm, out_hbm, row_f, bufa, bufb, hists)


def kernel(inputs):
    return _sort_kernel(inputs)


# speedup vs baseline: 1.3494x; 1.3494x over previous
import functools

import jax
import jax.numpy as jnp
from jax import lax
from jax.experimental import pallas as pl
from jax.experimental.pallas import tpu as pltpu
from jax.experimental.pallas import tpu_sc as plsc

ROWS, N = 128, 32768
NC, NS = 2, 16
NW = NC * NS
RPW = ROWS // NW
LANES = 16
CHUNK = N // LANES
SUB = 4
SUBLEN = CHUNK // SUB
NBINS = 256
FMASK = 0x7FFFFFFF


def _sort_body(in_hbm, out_hbm, row_f, bufa, bufb, hists):
    wid = lax.axis_index("s") * NC + lax.axis_index("c")
    lane = lax.iota(jnp.int32, LANES)
    base_idx = lane * CHUNK
    sub_base = [base_idx + j * SUBLEN for j in range(SUB)]
    ones = jnp.ones((LANES,), jnp.int32)
    zeros = jnp.zeros((LANES,), jnp.int32)

    def load_key_f32(idx):
        x = plsc.load_gather(row_f, [idx])
        u = plsc.bitcast(x, jnp.int32)
        return jnp.where(u < 0, u, u ^ FMASK)

    def store_i32(dst, pos, key):
        plsc.store_scatter(dst, [pos], key)

    def store_f32(dst, pos, key):
        v = jnp.where(key < 0, key, key ^ FMASK)
        plsc.store_scatter(dst, [pos], plsc.bitcast(v, jnp.float32))

    def one_pass(load_key, store_val, dst, shift):
        @plsc.parallel_loop(0, NBINS, unroll=2)
        def zero_body(b):
            for j in range(SUB):
                hists[j][pl.ds(b * LANES, LANES)] = zeros

        @plsc.parallel_loop(0, SUBLEN, unroll=2)
        def hist_body(i):
            for j in range(SUB):
                key = load_key(sub_base[j] + i)
                d = lax.shift_right_logical(key, shift) & 0xFF
                flat = (d << 4) | lane
                plsc.addupdate_scatter(hists[j], [flat], ones)

        def scan_body(b, carry):
            sl = pl.ds(b * LANES, LANES)
            v = [hists[j][sl] for j in range(SUB)]
            t = v[0] + v[1] + v[2] + v[3]
            o = plsc.cumsum(t) - t + carry
            for j in range(SUB):
                hists[j][sl] = o
                o = o + v[j]
            return carry + jnp.sum(t)

        lax.fori_loop(0, NBINS, scan_body, jnp.int32(0))

        def perm_body(i, _):
            for j in range(SUB):
                key = load_key(sub_base[j] + i)
                d = lax.shift_right_logical(key, shift) & 0xFF
                flat = (d << 4) | lane
                pos = plsc.load_gather(hists[j], [flat])
                plsc.addupdate_scatter(hists[j], [flat], ones)
                store_val(dst, pos, key)
            return 0

        lax.fori_loop(0, SUBLEN, perm_body, 0)

    def row_body(r, _):
        row = wid * RPW + r
        pltpu.sync_copy(in_hbm.at[row], row_f)
        one_pass(load_key_f32, store_i32, bufa, 0)
        one_pass(lambda idx: plsc.load_gather(bufa, [idx]), store_i32, bufb, 8)
        one_pass(lambda idx: plsc.load_gather(bufb, [idx]), store_i32, bufa, 16)
        one_pass(lambda idx: plsc.load_gather(bufa, [idx]), store_f32, row_f, 24)
        pltpu.sync_copy(row_f, out_hbm.at[row])
        return 0

    lax.fori_loop(0, RPW, row_body, 0)


@functools.partial(
    pl.kernel,
    out_type=jax.ShapeDtypeStruct((ROWS, N), jnp.float32),
    mesh=plsc.VectorSubcoreMesh(core_axis_name="c", subcore_axis_name="s"),
    scratch_types=[
        pltpu.VMEM((N,), jnp.float32),
        pltpu.VMEM((N,), jnp.int32),
        pltpu.VMEM((N,), jnp.int32),
        [pltpu.VMEM((NBINS * LANES,), jnp.int32) for _ in range(SUB)],
    ],
    compiler_params=pltpu.CompilerParams(needs_layout_passes=False),
)
def _sort_kernel(in_hbm, out_hbm, row_f, bufa, bufb, hists):
    _sort_body(in_hbm, out_hbm, row_f, bufa, bufb, hists)


def kernel(inputs):
    return _sort_kernel(inputs)


# speedup vs baseline: 1.3644x; 1.0111x over previous
import functools

import jax
import jax.numpy as jnp
from jax import lax
from jax.experimental import pallas as pl
from jax.experimental.pallas import tpu as pltpu
from jax.experimental.pallas import tpu_sc as plsc

ROWS, N = 128, 32768
NC, NS = 2, 16
NW = NC * NS
RPW = ROWS // NW
LANES = 16
CHUNK = N // LANES
SUB = 4
SUBLEN = CHUNK // SUB
NBINS = 256
FMASK = 0x7FFFFFFF


def _sort_body(in_hbm, out_hbm, row_f, bufa, bufb, hists):
    wid = lax.axis_index("s") * NC + lax.axis_index("c")
    lane = lax.iota(jnp.int32, LANES)
    base_idx = lane * CHUNK
    sub_base = [base_idx + j * SUBLEN for j in range(SUB)]
    ones = jnp.ones((LANES,), jnp.int32)
    zeros = jnp.zeros((LANES,), jnp.int32)

    def load_key_f32(idx):
        x = plsc.load_gather(row_f, [idx])
        u = plsc.bitcast(x, jnp.int32)
        return jnp.where(u < 0, u, u ^ FMASK)

    def store_i32(dst, pos, key):
        plsc.store_scatter(dst, [pos], key)

    def store_f32(dst, pos, key):
        v = jnp.where(key < 0, key, key ^ FMASK)
        plsc.store_scatter(dst, [pos], plsc.bitcast(v, jnp.float32))

    def one_pass(load_key, store_val, dst, shift):
        @plsc.parallel_loop(0, NBINS, unroll=2)
        def zero_body(b):
            for j in range(SUB):
                hists[j][pl.ds(b * LANES, LANES)] = zeros

        @plsc.parallel_loop(0, SUBLEN, unroll=2)
        def hist_body(i):
            for j in range(SUB):
                key = load_key(sub_base[j] + i)
                d = lax.shift_right_logical(key, shift) & 0xFF
                flat = (d << 4) | lane
                plsc.addupdate_scatter(hists[j], [flat], ones)

        def scan_body(b, carry):
            sl = pl.ds(b * LANES, LANES)
            v = [hists[j][sl] for j in range(SUB)]
            t = v[0] + v[1] + v[2] + v[3]
            o = plsc.cumsum(t) - t + carry
            for j in range(SUB):
                hists[j][sl] = o
                o = o + v[j]
            return carry + jnp.sum(t)

        lax.fori_loop(0, NBINS, scan_body, jnp.int32(0), unroll=4)

        def perm_body(i, _):
            for j in range(SUB):
                key = load_key(sub_base[j] + i)
                d = lax.shift_right_logical(key, shift) & 0xFF
                flat = (d << 4) | lane
                pos = plsc.load_gather(hists[j], [flat])
                plsc.addupdate_scatter(hists[j], [flat], ones)
                store_val(dst, pos, key)
            return 0

        lax.fori_loop(0, SUBLEN, perm_body, 0, unroll=4)

    def row_body(r, _):
        row = wid * RPW + r
        pltpu.sync_copy(in_hbm.at[row], row_f)
        one_pass(load_key_f32, store_i32, bufa, 0)
        one_pass(lambda idx: plsc.load_gather(bufa, [idx]), store_i32, bufb, 8)
        one_pass(lambda idx: plsc.load_gather(bufb, [idx]), store_i32, bufa, 16)
        one_pass(lambda idx: plsc.load_gather(bufa, [idx]), store_f32, row_f, 24)
        pltpu.sync_copy(row_f, out_hbm.at[row])
        return 0

    lax.fori_loop(0, RPW, row_body, 0)


@functools.partial(
    pl.kernel,
    out_type=jax.ShapeDtypeStruct((ROWS, N), jnp.float32),
    mesh=plsc.VectorSubcoreMesh(core_axis_name="c", subcore_axis_name="s"),
    scratch_types=[
        pltpu.VMEM((N,), jnp.float32),
        pltpu.VMEM((N,), jnp.int32),
        pltpu.VMEM((N,), jnp.int32),
        [pltpu.VMEM((NBINS * LANES,), jnp.int32) for _ in range(SUB)],
    ],
    compiler_params=pltpu.CompilerParams(needs_layout_passes=False),
)
def _sort_kernel(in_hbm, out_hbm, row_f, bufa, bufb, hists):
    _sort_body(in_hbm, out_hbm, row_f, bufa, bufb, hists)


def kernel(inputs):
    return _sort_kernel(inputs)


# speedup vs baseline: 2.5230x; 1.8492x over previous
import functools

import jax
import jax.numpy as jnp
from jax import lax
from jax.experimental import pallas as pl
from jax.experimental.pallas import tpu as pltpu
from jax.experimental.pallas import tpu_sc as plsc

ROWS, N = 128, 32768
NC, NS = 2, 16
NW = NC * NS
RPW = ROWS // NW
LANES = 16
NVEC = N // LANES
NPASS = 8
FMASK = 0x7FFFFFFF

_GDN = jax.lax.GatherDimensionNumbers(
    offset_dims=(), collapsed_slice_dims=(0,), start_index_map=(0,)
)


def _vperm(v, idx):
    return jax.lax.gather(
        v, idx[:, None], _GDN, slice_sizes=(1,),
        mode=jax.lax.GatherScatterMode.PROMISE_IN_BOUNDS,
    )


def _sort_body(in_hbm, out_hbm, bufa, bufb, hist):
    wid = lax.axis_index("s") * NC + lax.axis_index("c")
    lane = lax.iota(jnp.int32, LANES)
    zeros = jnp.zeros((LANES,), jnp.int32)

    def fwd_key(u):
        return jnp.where(u < 0, u, u ^ FMASK)

    def one_pass(src, dst, shift, first, last_pass):
        def digits(i):
            v = src[pl.ds(i * LANES, LANES)]
            key = fwd_key(v) if first else v
            d = lax.shift_right_logical(key, shift) & 0xF
            return key, d

        @plsc.parallel_loop(0, NVEC, carry=zeros)
        def tot(i, acc):
            _, d = digits(i)
            occ, lastm = plsc.scan_count(d)
            hist[pl.ds(i * LANES, LANES)] = zeros
            plsc.store_scatter(hist, [i * LANES + d], occ, mask=lastm)
            return acc + hist[pl.ds(i * LANES, LANES)]

        base = plsc.cumsum(tot) - tot

        @plsc.parallel_loop(0, NVEC, carry=zeros)
        def _run(i, run):
            key, d = digits(i)
            h = hist[pl.ds(i * LANES, LANES)]
            occ, _ = plsc.scan_count(d)
            pos = _vperm(run + base, d) + occ - 1
            out = jnp.where(key < 0, key, key ^ FMASK) if last_pass else key
            plsc.store_scatter(dst, [pos], out)
            return run + h

    def row_body(r, _):
        row = wid * RPW + r
        pltpu.sync_copy(in_hbm.at[row], bufa)
        for p in range(NPASS):
            src, dst = (bufa, bufb) if p % 2 == 0 else (bufb, bufa)
            one_pass(src, dst, p * 4, first=(p == 0), last_pass=(p == NPASS - 1))
        pltpu.sync_copy(bufa, out_hbm.at[row])
        return 0

    lax.fori_loop(0, RPW, row_body, 0)


@functools.partial(
    pl.kernel,
    out_type=jax.ShapeDtypeStruct((ROWS, N), jnp.int32),
    mesh=plsc.VectorSubcoreMesh(core_axis_name="c", subcore_axis_name="s"),
    scratch_types=[
        pltpu.VMEM((N,), jnp.int32),
        pltpu.VMEM((N,), jnp.int32),
        pltpu.VMEM((N,), jnp.int32),
    ],
    compiler_params=pltpu.CompilerParams(needs_layout_passes=False),
)
def _sort_kernel(in_hbm, out_hbm, bufa, bufb, hist):
    _sort_body(in_hbm, out_hbm, bufa, bufb, hist)


def kernel(inputs):
    raw = jax.lax.bitcast_convert_type(inputs, jnp.int32)
    out = _sort_kernel(raw)
    return jax.lax.bitcast_convert_type(out, jnp.float32)


# speedup vs baseline: 4.2745x; 1.6942x over previous
import functools

import jax
import jax.numpy as jnp
from jax import lax
from jax.experimental import pallas as pl
from jax.experimental.pallas import tpu as pltpu
from jax.experimental.pallas import tpu_sc as plsc

ROWS, N = 128, 32768
NC, NS = 2, 16
NW = NC * NS
RPW = ROWS // NW
LANES = 16
NVEC = N // LANES
NPASS = 8
FMASK = 0x7FFFFFFF

_GDN = jax.lax.GatherDimensionNumbers(
    offset_dims=(), collapsed_slice_dims=(0,), start_index_map=(0,)
)


def _vperm(v, idx):
    return jax.lax.gather(
        v, idx[:, None], _GDN, slice_sizes=(1,),
        mode=jax.lax.GatherScatterMode.PROMISE_IN_BOUNDS,
    )


def _sort_body(in_hbm, out_hbm, bufa, bufb, hist):
    wid = lax.axis_index("s") * NC + lax.axis_index("c")
    lane = lax.iota(jnp.int32, LANES)
    zeros = jnp.zeros((LANES,), jnp.int32)

    def fwd_key(u):
        return jnp.where(u < 0, u, u ^ FMASK)

    def one_pass(src, dst, shift, first, last_pass):
        def digits(i):
            v = src[pl.ds(i * LANES, LANES)]
            key = fwd_key(v) if first else v
            if shift == 0:
                d = key & 0xF
            elif shift == 32 - 4:
                d = lax.shift_right_logical(key, shift)
            else:
                d = lax.shift_right_logical(key, shift) & 0xF
            return key, d

        @plsc.parallel_loop(0, NVEC, unroll=2, carry=zeros)
        def tot(i, acc):
            _, d = digits(i)
            occ, lastm = plsc.scan_count(d)
            hist[pl.ds(i * LANES, LANES)] = zeros
            plsc.store_scatter(hist, [i * LANES + d], occ, mask=lastm)
            return acc + hist[pl.ds(i * LANES, LANES)]

        base = plsc.cumsum(tot) - tot - 1

        @plsc.parallel_loop(0, NVEC, unroll=2, carry=base)
        def _run(i, run):
            key, d = digits(i)
            h = hist[pl.ds(i * LANES, LANES)]
            occ, _ = plsc.scan_count(d)
            pos = _vperm(run, d) + occ
            out = jnp.where(key < 0, key, key ^ FMASK) if last_pass else key
            plsc.store_scatter(dst, [pos], out)
            return run + h

    def row_body(r, _):
        row = wid * RPW + r
        pltpu.sync_copy(in_hbm.at[row], bufa)
        for p in range(NPASS):
            src, dst = (bufa, bufb) if p % 2 == 0 else (bufb, bufa)
            one_pass(src, dst, p * 4, first=(p == 0), last_pass=(p == NPASS - 1))
        pltpu.sync_copy(bufa, out_hbm.at[row])
        return 0

    lax.fori_loop(0, RPW, row_body, 0)


@functools.partial(
    pl.kernel,
    out_type=jax.ShapeDtypeStruct((ROWS, N), jnp.int32),
    mesh=plsc.VectorSubcoreMesh(core_axis_name="c", subcore_axis_name="s"),
    scratch_types=[
        pltpu.VMEM((N,), jnp.int32),
        pltpu.VMEM((N,), jnp.int32),
        pltpu.VMEM((N,), jnp.int32),
    ],
    compiler_params=pltpu.CompilerParams(needs_layout_passes=False),
)
def _sort_kernel(in_hbm, out_hbm, bufa, bufb, hist):
    _sort_body(in_hbm, out_hbm, bufa, bufb, hist)


def kernel(inputs):
    raw = jax.lax.bitcast_convert_type(inputs, jnp.int32)
    out = _sort_kernel(raw)
    return jax.lax.bitcast_convert_type(out, jnp.float32)


# speedup vs baseline: 5.0615x; 1.1841x over previous
import functools

import jax
import jax.numpy as jnp
from jax import lax
from jax.experimental import pallas as pl
from jax.experimental.pallas import tpu as pltpu
from jax.experimental.pallas import tpu_sc as plsc

ROWS, N = 128, 32768
NC, NS = 2, 16
NW = NC * NS
RPW = ROWS // NW
LANES = 16
NVEC = N // LANES
NPASS = 8
FMASK = 0x7FFFFFFF

_GDN = jax.lax.GatherDimensionNumbers(
    offset_dims=(), collapsed_slice_dims=(0,), start_index_map=(0,)
)


def _vperm(v, idx):
    return jax.lax.gather(
        v, idx[:, None], _GDN, slice_sizes=(1,),
        mode=jax.lax.GatherScatterMode.PROMISE_IN_BOUNDS,
    )


def _sort_body(in_hbm, out_hbm, bufa, bufb, hist):
    wid = lax.axis_index("s") * NC + lax.axis_index("c")
    lane = lax.iota(jnp.int32, LANES)
    zeros = jnp.zeros((LANES,), jnp.int32)

    def fwd_key(u):
        return jnp.where(u < 0, u, u ^ FMASK)

    def one_pass(src, dst, shift, first, last_pass):
        def digits(i):
            v = src[pl.ds(i * LANES, LANES)]
            key = fwd_key(v) if first else v
            if shift == 0:
                d = key & 0xF
            elif shift == 32 - 4:
                d = lax.shift_right_logical(key, shift)
            else:
                d = lax.shift_right_logical(key, shift) & 0xF
            return key, d

        @plsc.parallel_loop(0, NVEC, unroll=4, carry=zeros)
        def tot(i, acc):
            _, d = digits(i)
            occ, lastm = plsc.scan_count(d)
            hist[pl.ds(i * LANES, LANES)] = zeros
            plsc.store_scatter(hist, [i * LANES + d], occ, mask=lastm)
            return acc + hist[pl.ds(i * LANES, LANES)]

        base = plsc.cumsum(tot) - tot - 1

        @plsc.parallel_loop(0, NVEC, unroll=4, carry=base)
        def _run(i, run):
            key, d = digits(i)
            h = hist[pl.ds(i * LANES, LANES)]
            occ, _ = plsc.scan_count(d)
            pos = _vperm(run, d) + occ
            out = jnp.where(key < 0, key, key ^ FMASK) if last_pass else key
            plsc.store_scatter(dst, [pos], out)
            return run + h

    def row_body(r, _):
        row = wid * RPW + r
        pltpu.sync_copy(in_hbm.at[row], bufa)
        for p in range(NPASS):
            src, dst = (bufa, bufb) if p % 2 == 0 else (bufb, bufa)
            one_pass(src, dst, p * 4, first=(p == 0), last_pass=(p == NPASS - 1))
        pltpu.sync_copy(bufa, out_hbm.at[row])
        return 0

    lax.fori_loop(0, RPW, row_body, 0)


@functools.partial(
    pl.kernel,
    out_type=jax.ShapeDtypeStruct((ROWS, N), jnp.int32),
    mesh=plsc.VectorSubcoreMesh(core_axis_name="c", subcore_axis_name="s"),
    scratch_types=[
        pltpu.VMEM((N,), jnp.int32),
        pltpu.VMEM((N,), jnp.int32),
        pltpu.VMEM((N,), jnp.int32),
    ],
    compiler_params=pltpu.CompilerParams(needs_layout_passes=False),
)
def _sort_kernel(in_hbm, out_hbm, bufa, bufb, hist):
    _sort_body(in_hbm, out_hbm, bufa, bufb, hist)


def kernel(inputs):
    raw = jax.lax.bitcast_convert_type(inputs, jnp.int32)
    out = _sort_kernel(raw)
    return jax.lax.bitcast_convert_type(out, jnp.float32)


# speedup vs baseline: 5.4477x; 1.0763x over previous
import functools

import jax
import jax.numpy as jnp
from jax import lax
from jax.experimental import pallas as pl
from jax.experimental.pallas import tpu as pltpu
from jax.experimental.pallas import tpu_sc as plsc

ROWS, N = 128, 32768
NC, NS = 2, 16
NW = NC * NS
RPW = ROWS // NW
LANES = 16
NVEC = N // LANES
NPASS = 8
FMASK = 0x7FFFFFFF

_GDN = jax.lax.GatherDimensionNumbers(
    offset_dims=(), collapsed_slice_dims=(0,), start_index_map=(0,)
)


def _vperm(v, idx):
    return jax.lax.gather(
        v, idx[:, None], _GDN, slice_sizes=(1,),
        mode=jax.lax.GatherScatterMode.PROMISE_IN_BOUNDS,
    )


def _sort_body(in_hbm, out_hbm, bufa, bufb, hist):
    wid = lax.axis_index("s") * NC + lax.axis_index("c")
    lane = lax.iota(jnp.int32, LANES)
    zeros = jnp.zeros((LANES,), jnp.int32)

    def fwd_key(u):
        return jnp.where(u < 0, u, u ^ FMASK)

    def one_pass(src, dst, shift, first, last_pass):
        def digits(i):
            v = src[pl.ds(i * LANES, LANES)]
            key = fwd_key(v) if first else v
            if shift == 0:
                d = key & 0xF
            elif shift == 32 - 4:
                d = lax.shift_right_logical(key, shift)
            else:
                d = lax.shift_right_logical(key, shift) & 0xF
            return key, d

        @plsc.parallel_loop(0, NVEC, unroll=8, carry=zeros)
        def tot(i, acc):
            _, d = digits(i)
            occ, lastm = plsc.scan_count(d)
            hist[pl.ds(i * LANES, LANES)] = zeros
            plsc.store_scatter(hist, [i * LANES + d], occ, mask=lastm)
            return acc + hist[pl.ds(i * LANES, LANES)]

        base = plsc.cumsum(tot) - tot - 1

        @plsc.parallel_loop(0, NVEC, unroll=8, carry=base)
        def _run(i, run):
            key, d = digits(i)
            h = hist[pl.ds(i * LANES, LANES)]
            occ, _ = plsc.scan_count(d)
            pos = _vperm(run, d) + occ
            out = jnp.where(key < 0, key, key ^ FMASK) if last_pass else key
            plsc.store_scatter(dst, [pos], out)
            return run + h

    def row_body(r, _):
        row = wid * RPW + r
        pltpu.sync_copy(in_hbm.at[row], bufa)
        for p in range(NPASS):
            src, dst = (bufa, bufb) if p % 2 == 0 else (bufb, bufa)
            one_pass(src, dst, p * 4, first=(p == 0), last_pass=(p == NPASS - 1))
        pltpu.sync_copy(bufa, out_hbm.at[row])
        return 0

    lax.fori_loop(0, RPW, row_body, 0)


@functools.partial(
    pl.kernel,
    out_type=jax.ShapeDtypeStruct((ROWS, N), jnp.int32),
    mesh=plsc.VectorSubcoreMesh(core_axis_name="c", subcore_axis_name="s"),
    scratch_types=[
        pltpu.VMEM((N,), jnp.int32),
        pltpu.VMEM((N,), jnp.int32),
        pltpu.VMEM((N,), jnp.int32),
    ],
    compiler_params=pltpu.CompilerParams(needs_layout_passes=False),
)
def _sort_kernel(in_hbm, out_hbm, bufa, bufb, hist):
    _sort_body(in_hbm, out_hbm, bufa, bufb, hist)


def kernel(inputs):
    raw = jax.lax.bitcast_convert_type(inputs, jnp.int32)
    out = _sort_kernel(raw)
    return jax.lax.bitcast_convert_type(out, jnp.float32)


# speedup vs baseline: 5.4920x; 1.0081x over previous
import functools

import jax
import jax.numpy as jnp
from jax import lax
from jax.experimental import pallas as pl
from jax.experimental.pallas import tpu as pltpu
from jax.experimental.pallas import tpu_sc as plsc

ROWS, N = 128, 32768
NC, NS = 2, 16
NW = NC * NS
RPW = ROWS // NW
LANES = 16
NVEC = N // LANES
NPASS = 8
FMASK = 0x7FFFFFFF

_GDN = jax.lax.GatherDimensionNumbers(
    offset_dims=(), collapsed_slice_dims=(0,), start_index_map=(0,)
)


def _vperm(v, idx):
    return jax.lax.gather(
        v, idx[:, None], _GDN, slice_sizes=(1,),
        mode=jax.lax.GatherScatterMode.PROMISE_IN_BOUNDS,
    )


def _sort_body(in_hbm, out_hbm, bufa, bufb, hist):
    wid = lax.axis_index("s") * NC + lax.axis_index("c")
    lane = lax.iota(jnp.int32, LANES)
    zeros = jnp.zeros((LANES,), jnp.int32)

    def fwd_key(u):
        return jnp.where(u < 0, u, u ^ FMASK)

    def one_pass(src, dst, shift, first, last_pass):
        def digits(i):
            v = src[pl.ds(i * LANES, LANES)]
            key = fwd_key(v) if first else v
            if shift == 0:
                d = key & 0xF
            elif shift == 32 - 4:
                d = lax.shift_right_logical(key, shift)
            else:
                d = lax.shift_right_logical(key, shift) & 0xF
            return key, d

        @plsc.parallel_loop(0, NVEC, unroll=16, carry=zeros)
        def tot(i, acc):
            _, d = digits(i)
            occ, lastm = plsc.scan_count(d)
            hist[pl.ds(i * LANES, LANES)] = zeros
            plsc.store_scatter(hist, [i * LANES + d], occ, mask=lastm)
            return acc + hist[pl.ds(i * LANES, LANES)]

        base = plsc.cumsum(tot) - tot - 1

        @plsc.parallel_loop(0, NVEC, unroll=16, carry=base)
        def _run(i, run):
            key, d = digits(i)
            h = hist[pl.ds(i * LANES, LANES)]
            occ, _ = plsc.scan_count(d)
            pos = _vperm(run, d) + occ
            out = jnp.where(key < 0, key, key ^ FMASK) if last_pass else key
            plsc.store_scatter(dst, [pos], out)
            return run + h

    def row_body(r, _):
        row = wid * RPW + r
        pltpu.sync_copy(in_hbm.at[row], bufa)
        for p in range(NPASS):
            src, dst = (bufa, bufb) if p % 2 == 0 else (bufb, bufa)
            one_pass(src, dst, p * 4, first=(p == 0), last_pass=(p == NPASS - 1))
        pltpu.sync_copy(bufa, out_hbm.at[row])
        return 0

    lax.fori_loop(0, RPW, row_body, 0)


@functools.partial(
    pl.kernel,
    out_type=jax.ShapeDtypeStruct((ROWS, N), jnp.int32),
    mesh=plsc.VectorSubcoreMesh(core_axis_name="c", subcore_axis_name="s"),
    scratch_types=[
        pltpu.VMEM((N,), jnp.int32),
        pltpu.VMEM((N,), jnp.int32),
        pltpu.VMEM((N,), jnp.int32),
    ],
    compiler_params=pltpu.CompilerParams(needs_layout_passes=False),
)
def _sort_kernel(in_hbm, out_hbm, bufa, bufb, hist):
    _sort_body(in_hbm, out_hbm, bufa, bufb, hist)


def kernel(inputs):
    raw = jax.lax.bitcast_convert_type(inputs, jnp.int32)
    out = _sort_kernel(raw)
    return jax.lax.bitcast_convert_type(out, jnp.float32)
